# baseline (speedup 1.0000x reference)
import functools
import jax
import jax.numpy as jnp
from jax import lax
from jax.experimental import pallas as pl
from jax.experimental.pallas import tpu as pltpu
from jax.experimental.pallas import tpu_sc as plsc

N = 10000
E = 320000
IN_CH = 128
OUT_CH = 4
HID = 2 * OUT_CH

NC = 2
NS = 16
NW = NC * NS
EPW = E // NW
B = 1000
J = EPW // B
PAD_N = 10112
RPT = PAD_N // NS

_mesh = lambda: plsc.VectorSubcoreMesh(core_axis_name="c", subcore_axis_name="s")


def _load_indices(edge_hbm, e0, src_v, dst_v, sem):
    for j in range(J):
        pltpu.async_copy(edge_hbm.at[0, pl.ds(e0 + j * B, B)], src_v.at[j], sem)
        pltpu.async_copy(edge_hbm.at[1, pl.ds(e0 + j * B, B)], dst_v.at[j], sem)
    for j in range(J):
        pltpu.make_async_copy(edge_hbm.at[0, pl.ds(e0, B)], src_v.at[j], sem).wait()
        pltpu.make_async_copy(edge_hbm.at[0, pl.ds(e0, B)], dst_v.at[j], sem).wait()


def _make_deg_pass():
    @functools.partial(
        pl.kernel,
        out_type=jax.ShapeDtypeStruct((NC, PAD_N, HID), jnp.float32),
        mesh=_mesh(),
        compiler_params=pltpu.CompilerParams(use_tc_tiling_on_sc=False),
        scratch_types=[
            pltpu.VMEM((J, B), jnp.int32),
            pltpu.VMEM((J, B), jnp.int32),
            pltpu.VMEM((B, HID), jnp.float32),
            pltpu.VMEM_SHARED((PAD_N, HID), jnp.float32),
            pltpu.SemaphoreType.DMA,
        ],
    )
    def deg_pass(edge_hbm, ones_hbm, zero_hbm, out_hbm,
                 src_v, dst_v, ones_v, acc_sh, sem):
        cid = lax.axis_index("c")
        sid = lax.axis_index("s")
        wid = sid * NC + cid
        row0 = sid * RPT

        pltpu.sync_copy(zero_hbm.at[pl.ds(row0, RPT)],
                        acc_sh.at[pl.ds(row0, RPT)])
        _load_indices(edge_hbm, wid * EPW, src_v, dst_v, sem)
        pltpu.sync_copy(ones_hbm, ones_v)
        plsc.subcore_barrier()

        def body(j, carry):
            pltpu.sync_copy(ones_v, acc_sh.at[dst_v.at[j]], add=True)
            return carry

        lax.fori_loop(0, J, body, 0)
        plsc.subcore_barrier()
        pltpu.sync_copy(acc_sh.at[pl.ds(row0, RPT)],
                        out_hbm.at[cid, pl.ds(row0, RPT)])

    return deg_pass


def _make_edge_pass(C):
    @functools.partial(
        pl.kernel,
        out_type=jax.ShapeDtypeStruct((NC, PAD_N, C), jnp.float32),
        mesh=_mesh(),
        compiler_params=pltpu.CompilerParams(use_tc_tiling_on_sc=False),
        scratch_types=[
            pltpu.VMEM((J, B), jnp.int32),
            pltpu.VMEM((J, B), jnp.int32),
            pltpu.VMEM((B, C), jnp.float32),
            pltpu.VMEM((B, C), jnp.float32),
            pltpu.VMEM((B, C), jnp.float32),
            pltpu.VMEM_SHARED((PAD_N, C), jnp.float32),
            pltpu.SemaphoreType.DMA,
            pltpu.SemaphoreType.DMA,
            pltpu.SemaphoreType.DMA,
            pltpu.SemaphoreType.DMA,
            pltpu.SemaphoreType.DMA,
        ],
    )
    def edge_pass(g_hbm, edge_hbm, zero_hbm, out_hbm,
                  src_v, dst_v, msg0, msg1, msg2, acc_sh,
                  sem0, sem1, sem2, ssem, semi):
        cid = lax.axis_index("c")
        sid = lax.axis_index("s")
        wid = sid * NC + cid
        row0 = sid * RPT

        pltpu.sync_copy(zero_hbm.at[pl.ds(row0, RPT)],
                        acc_sh.at[pl.ds(row0, RPT)])
        _load_indices(edge_hbm, wid * EPW, src_v, dst_v, semi)
        plsc.subcore_barrier()

        bufs = (msg0, msg1, msg2)
        sems = (sem0, sem1, sem2)
        pltpu.async_copy(g_hbm.at[src_v.at[0]], bufs[0], sems[0])
        pltpu.async_copy(g_hbm.at[src_v.at[1]], bufs[1], sems[1])
        n_sc = 0
        for j in range(J):
            b = j % 3
            if j + 2 < J:
                if j >= 1:
                    pltpu.make_async_copy(
                        bufs[(j - 1) % 3],
                        acc_sh.at[dst_v.at[j - 1]], ssem).wait()
                    n_sc -= 1
                pltpu.async_copy(g_hbm.at[src_v.at[j + 2]],
                                 bufs[(j + 2) % 3], sems[(j + 2) % 3])
            pltpu.make_async_copy(g_hbm.at[src_v.at[j]], bufs[b],
                                  sems[b]).wait()
            pltpu.async_copy(bufs[b], acc_sh.at[dst_v.at[j]], ssem,
                             add=True)
            n_sc += 1
        for j in range(J - n_sc, J):
            pltpu.make_async_copy(bufs[j % 3], acc_sh.at[dst_v.at[j]],
                                  ssem).wait()
        plsc.subcore_barrier()
        pltpu.sync_copy(acc_sh.at[pl.ds(row0, RPT)],
                        out_hbm.at[cid, pl.ds(row0, RPT)])

    return edge_pass


_deg_pass = _make_deg_pass()
_edge_pass_h = _make_edge_pass(HID)


NV = N * HID // 128
NVP = PAD_N * HID // 128


def _tc1(x_ref, w1_ref, degp_ref, g1_ref, dinv_ref):
    deg = degp_ref[0, 0:NV] + degp_ref[1, 0:NV] + 1.0
    dinv = lax.rsqrt(deg)
    h1 = jnp.dot(x_ref[...], w1_ref[...],
                 preferred_element_type=jnp.float32)
    g1_ref[...] = h1 * dinv
    dinv_ref[...] = dinv


def _tc2(dinv_ref, g1_ref, a1_ref, w2_ref, b1_ref, g2_ref):
    dinv = dinv_ref[...]
    out1 = (dinv * (a1_ref[0, 0:NV] + a1_ref[1, 0:NV] + g1_ref[...])
            + b1_ref[...])
    h2 = jnp.dot(jnp.maximum(out1, 0.0), w2_ref[...],
                 preferred_element_type=jnp.float32)
    g2_ref[...] = h2 * dinv


def _tc3(dinv_ref, g2_ref, a2_ref, b2_ref, out_ref):
    out_ref[...] = (dinv_ref[...]
                    * (a2_ref[0, 0:NV] + a2_ref[1, 0:NV] + g2_ref[...])
                    + b2_ref[...])


def kernel(x, edge_index, W1, b1, W2, b2):
    f32 = jnp.float32
    edges = edge_index.astype(jnp.int32)
    ones_b = jnp.ones((B, HID), f32)
    zeros_h = jnp.zeros((PAD_N, HID), f32)
    W2p = jnp.zeros((HID, HID), f32).at[:, :OUT_CH].set(W2)
    eye16 = jnp.eye(16, dtype=f32)
    W1big = jnp.kron(eye16, W1)
    W2big = jnp.kron(eye16, W2p)
    b1t = jnp.tile(b1, 16).reshape(1, 128)
    b2t = jnp.tile(jnp.zeros((HID,), f32).at[:OUT_CH].set(b2),
                   16).reshape(1, 128)

    degp = _deg_pass(edges, ones_b, zeros_h)

    g1v, dinvv = pl.pallas_call(
        _tc1,
        out_shape=[jax.ShapeDtypeStruct((NV, 128), f32),
                   jax.ShapeDtypeStruct((NV, 128), f32)],
    )(x.reshape(NV, 16 * IN_CH), W1big, degp.reshape(NC, NVP, 128))

    a1 = _edge_pass_h(g1v.reshape(N, HID), edges, zeros_h)

    g2v = pl.pallas_call(
        _tc2,
        out_shape=jax.ShapeDtypeStruct((NV, 128), f32),
    )(dinvv, g1v, a1.reshape(NC, NVP, 128), W2big, b1t)

    a2 = _edge_pass_h(g2v.reshape(N, HID), edges, zeros_h)

    outv = pl.pallas_call(
        _tc3,
        out_shape=jax.ShapeDtypeStruct((NV, 128), f32),
    )(dinvv, g2v, a2.reshape(NC, NVP, 128), b2t)

    return outv.reshape(N, HID)[:, :OUT_CH]


# speedup vs baseline: 1.0036x; 1.0036x over previous
import functools
import jax
import jax.numpy as jnp
from jax import lax
from jax.experimental import pallas as pl
from jax.experimental.pallas import tpu as pltpu
from jax.experimental.pallas import tpu_sc as plsc

N = 10000
E = 320000
IN_CH = 128
OUT_CH = 4
HID = 2 * OUT_CH

NC = 2
NS = 16
NW = NC * NS
EPW = E // NW
B = 1000
J = EPW // B
PAD_N = 10112
RPT = PAD_N // NS

_mesh = lambda: plsc.VectorSubcoreMesh(core_axis_name="c", subcore_axis_name="s")


def _load_indices(edge_hbm, e0, src_v, dst_v, sem):
    for j in range(J):
        pltpu.async_copy(edge_hbm.at[0, pl.ds(e0 + j * B, B)], src_v.at[j], sem)
        pltpu.async_copy(edge_hbm.at[1, pl.ds(e0 + j * B, B)], dst_v.at[j], sem)
    for j in range(J):
        pltpu.make_async_copy(edge_hbm.at[0, pl.ds(e0, B)], src_v.at[j], sem).wait()
        pltpu.make_async_copy(edge_hbm.at[0, pl.ds(e0, B)], dst_v.at[j], sem).wait()


def _make_deg_pass():
    @functools.partial(
        pl.kernel,
        out_type=jax.ShapeDtypeStruct((NC, PAD_N, HID), jnp.float32),
        mesh=_mesh(),
        compiler_params=pltpu.CompilerParams(use_tc_tiling_on_sc=False),
        scratch_types=[
            pltpu.VMEM((J, B), jnp.int32),
            pltpu.VMEM((J, B), jnp.int32),
            pltpu.VMEM((B, HID), jnp.float32),
            pltpu.VMEM_SHARED((PAD_N, HID), jnp.float32),
            pltpu.SemaphoreType.DMA,
        ],
    )
    def deg_pass(edge_hbm, ones_hbm, zero_hbm, out_hbm,
                 src_v, dst_v, ones_v, acc_sh, sem):
        cid = lax.axis_index("c")
        sid = lax.axis_index("s")
        wid = sid * NC + cid
        row0 = sid * RPT

        pltpu.sync_copy(zero_hbm.at[pl.ds(row0, RPT)],
                        acc_sh.at[pl.ds(row0, RPT)])
        _load_indices(edge_hbm, wid * EPW, src_v, dst_v, sem)
        pltpu.sync_copy(ones_hbm, ones_v)
        plsc.subcore_barrier()

        def body(j, carry):
            pltpu.sync_copy(ones_v, acc_sh.at[dst_v.at[j]], add=True)
            return carry

        lax.fori_loop(0, J, body, 0)
        plsc.subcore_barrier()
        pltpu.sync_copy(acc_sh.at[pl.ds(row0, RPT)],
                        out_hbm.at[cid, pl.ds(row0, RPT)])

    return deg_pass


def _make_edge_pass(C):
    @functools.partial(
        pl.kernel,
        out_type=jax.ShapeDtypeStruct((NC, PAD_N, C), jnp.float32),
        mesh=_mesh(),
        compiler_params=pltpu.CompilerParams(use_tc_tiling_on_sc=False),
        scratch_types=[
            pltpu.VMEM((J, B), jnp.int32),
            pltpu.VMEM((J, B), jnp.int32),
            pltpu.VMEM((B, C), jnp.float32),
            pltpu.VMEM((B, C), jnp.float32),
            pltpu.VMEM_SHARED((PAD_N, C), jnp.float32),
            pltpu.SemaphoreType.DMA,
            pltpu.SemaphoreType.DMA,
            pltpu.SemaphoreType.DMA,
        ],
    )
    def edge_pass(g_hbm, edge_hbm, zero_hbm, out_hbm,
                  src_v, dst_v, msg0, msg1, acc_sh, sem0, sem1, semi):
        cid = lax.axis_index("c")
        sid = lax.axis_index("s")
        wid = sid * NC + cid
        row0 = sid * RPT

        pltpu.sync_copy(zero_hbm.at[pl.ds(row0, RPT)],
                        acc_sh.at[pl.ds(row0, RPT)])
        _load_indices(edge_hbm, wid * EPW, src_v, dst_v, semi)
        plsc.subcore_barrier()

        pltpu.async_copy(g_hbm.at[src_v.at[0]], msg0, sem0)

        def body(k, carry):
            j0 = 2 * k
            pltpu.async_copy(g_hbm.at[src_v.at[j0 + 1]], msg1, sem1)
            pltpu.make_async_copy(g_hbm.at[src_v.at[j0]], msg0, sem0).wait()
            pltpu.sync_copy(msg0, acc_sh.at[dst_v.at[j0]], add=True)

            @pl.when(j0 + 2 < J)
            def _():
                pltpu.async_copy(g_hbm.at[src_v.at[j0 + 2]], msg0, sem0)

            pltpu.make_async_copy(g_hbm.at[src_v.at[j0 + 1]], msg1,
                                  sem1).wait()
            pltpu.sync_copy(msg1, acc_sh.at[dst_v.at[j0 + 1]], add=True)
            return carry

        lax.fori_loop(0, J // 2, body, 0)
        plsc.subcore_barrier()
        pltpu.sync_copy(acc_sh.at[pl.ds(row0, RPT)],
                        out_hbm.at[cid, pl.ds(row0, RPT)])

    return edge_pass


_deg_pass = _make_deg_pass()
_edge_pass_h = _make_edge_pass(HID)


NV = N * HID // 128
NVP = PAD_N * HID // 128


def _tc1(x_ref, w1_ref, degp_ref, g1_ref, dinv_ref):
    deg = degp_ref[0, 0:NV] + degp_ref[1, 0:NV] + 1.0
    dinv = lax.rsqrt(deg)
    h1 = jnp.dot(x_ref[...], w1_ref[...],
                 preferred_element_type=jnp.float32)
    g1_ref[...] = h1 * dinv
    dinv_ref[...] = dinv


def _tc2(dinv_ref, g1_ref, a1_ref, w2_ref, b1_ref, g2_ref):
    dinv = dinv_ref[...]
    out1 = (dinv * (a1_ref[0, 0:NV] + a1_ref[1, 0:NV] + g1_ref[...])
            + b1_ref[...])
    h2 = jnp.dot(jnp.maximum(out1, 0.0), w2_ref[...],
                 preferred_element_type=jnp.float32)
    g2_ref[...] = h2 * dinv


def _tc3(dinv_ref, g2_ref, a2_ref, b2_ref, out_ref):
    out_ref[...] = (dinv_ref[...]
                    * (a2_ref[0, 0:NV] + a2_ref[1, 0:NV] + g2_ref[...])
                    + b2_ref[...])


def kernel(x, edge_index, W1, b1, W2, b2):
    f32 = jnp.float32
    edges = edge_index.astype(jnp.int32)
    ones_b = jnp.ones((B, HID), f32)
    zeros_h = jnp.zeros((PAD_N, HID), f32)
    W2p = jnp.zeros((HID, HID), f32).at[:, :OUT_CH].set(W2)
    eye16 = jnp.eye(16, dtype=f32)
    W1big = jnp.kron(eye16, W1)
    W2big = jnp.kron(eye16, W2p)
    b1t = jnp.tile(b1, 16).reshape(1, 128)
    b2t = jnp.tile(jnp.zeros((HID,), f32).at[:OUT_CH].set(b2),
                   16).reshape(1, 128)

    degp = _deg_pass(edges, ones_b, zeros_h)

    g1v, dinvv = pl.pallas_call(
        _tc1,
        out_shape=[jax.ShapeDtypeStruct((NV, 128), f32),
                   jax.ShapeDtypeStruct((NV, 128), f32)],
    )(x.reshape(NV, 16 * IN_CH), W1big, degp.reshape(NC, NVP, 128))

    a1 = _edge_pass_h(g1v.reshape(N, HID), edges, zeros_h)

    g2v = pl.pallas_call(
        _tc2,
        out_shape=jax.ShapeDtypeStruct((NV, 128), f32),
    )(dinvv, g1v, a1.reshape(NC, NVP, 128), W2big, b1t)

    a2 = _edge_pass_h(g2v.reshape(N, HID), edges, zeros_h)

    outv = pl.pallas_call(
        _tc3,
        out_shape=jax.ShapeDtypeStruct((NV, 128), f32),
    )(dinvv, g2v, a2.reshape(NC, NVP, 128), b2t)

    return outv.reshape(N, HID)[:, :OUT_CH]
